# baseline (speedup 1.0000x reference)
import functools

import jax
import jax.numpy as jnp
from jax import lax
from jax.experimental import pallas as pl
from jax.experimental.pallas import tpu as pltpu
from jax.experimental.pallas import tpu_sc as plsc

E = 64
K = 2
D = 768
N = 4096
CAP = 160
CAPP = CAP + 1
NSLOT = E * CAPP
PAD_ROW = N

B = 512
NB = N // B

NC = 2
NS = 16
NW = NC * NS
L = 16

TPT = N // NW
CCH = 16


def _router_body(x_ref, wg_ref, tril_ref, d0_ref, d1_ref, s0_ref, s1_ref,
                 g0_ref, g1_ref, carry_ref):
    i = pl.program_id(0)

    @pl.when(i == 0)
    def _():
        carry_ref[...] = jnp.zeros_like(carry_ref)

    x = x_ref[...]
    wg = wg_ref[...]
    logits = jnp.dot(x, wg, preferred_element_type=jnp.float32)

    re = lax.broadcasted_iota(jnp.int32, (E, E), 0)
    ce = lax.broadcasted_iota(jnp.int32, (E, E), 1)
    ut = jnp.where(re < ce, 1.0, 0.0)
    iota_col = lax.broadcasted_iota(jnp.int32, (E, 1), 0).astype(jnp.float32)
    ones_col = jnp.full((E, 1), 1.0, jnp.float32)

    def first_max(v):
        m = jnp.max(v, axis=1, keepdims=True)
        eq = jnp.where(v == m, 1.0, 0.0)
        before = jnp.dot(eq, ut, preferred_element_type=jnp.float32)
        oh = eq * jnp.where(before == 0.0, 1.0, 0.0)
        idx = jnp.dot(oh, iota_col, preferred_element_type=jnp.float32)
        return m, oh, idx

    m0, oh0, idx0 = first_max(logits)
    masked = jnp.where(oh0 > 0.0, -jnp.inf, logits)
    m1, oh1, idx1 = first_max(masked)

    t = jnp.exp(m1 - m0)
    g0 = 1.0 / (1.0 + t)
    g1 = 1.0 - g0

    ohsum = oh0 + oh1
    cnt_before = (jnp.dot(tril_ref[...], ohsum,
                          preferred_element_type=jnp.float32)
                  + carry_ref[...])
    carry_ref[...] = carry_ref[...] + jnp.sum(ohsum, axis=0, keepdims=True)

    pos0 = jnp.dot(oh0 * cnt_before, ones_col,
                   preferred_element_type=jnp.float32)
    pos1 = jnp.dot(oh1 * cnt_before, ones_col,
                   preferred_element_type=jnp.float32)
    keep0 = pos0 < CAP
    keep1 = pos1 < CAP
    base0 = idx0 * CAPP
    base1 = idx1 * CAPP
    d0 = base0 + jnp.minimum(pos0, CAP - 1.0)
    d1 = base1 + jnp.minimum(pos1, CAP - 1.0)

    d0_ref[...] = d0.astype(jnp.int32)
    d1_ref[...] = d1.astype(jnp.int32)
    s0_ref[...] = (base0 + jnp.minimum(pos0, float(CAP))).astype(jnp.int32)
    s1_ref[...] = (base1 + jnp.minimum(pos1, float(CAP))).astype(jnp.int32)
    g0_ref[...] = g0 * keep0.astype(jnp.float32)
    g1_ref[...] = g1 * keep1.astype(jnp.float32)


def _router(x, Wg, tril):
    blk = pl.BlockSpec((B, 1), lambda i: (i, 0))
    iod = jax.ShapeDtypeStruct((N, 1), jnp.int32)
    fod = jax.ShapeDtypeStruct((N, 1), jnp.float32)
    return pl.pallas_call(
        _router_body,
        grid=(NB,),
        in_specs=[
            pl.BlockSpec((B, D), lambda i: (i, 0)),
            pl.BlockSpec((D, E), lambda i: (0, 0)),
            pl.BlockSpec((B, B), lambda i: (0, 0)),
        ],
        out_specs=[blk] * 6,
        out_shape=[iod, iod, iod, iod, fod, fod],
        scratch_shapes=[pltpu.VMEM((1, E), jnp.float32)],
    )(x, Wg, tril)


DCH = 64
DNCH = (N // NW) // DCH


def _dispatch_body(x_hbm, s0_hbm, s1_hbm, disp_hbm,
                   idx_v, xb0_v, xb1_v, sem_in, sem_out):
    wid = lax.axis_index("s") * NC + lax.axis_index("c")
    base = wid * (N // NW)

    for ch in range(DNCH):
        t0 = base + ch * DCH
        pltpu.sync_copy(s0_hbm.at[pl.ds(t0, DCH)], idx_v.at[2 * ch])
        pltpu.sync_copy(s1_hbm.at[pl.ds(t0, DCH)], idx_v.at[2 * ch + 1])

    bufs = [xb0_v, xb1_v]
    cp = pltpu.async_copy(x_hbm.at[pl.ds(base, DCH)], bufs[0], sem_in)
    cp.wait()
    for ch in range(DNCH):
        buf = bufs[ch % 2]
        if ch + 1 < DNCH:
            nxt = pltpu.async_copy(
                x_hbm.at[pl.ds(base + (ch + 1) * DCH, DCH)],
                bufs[(ch + 1) % 2], sem_in)
        o0 = pltpu.async_copy(buf, disp_hbm.at[idx_v.at[2 * ch]], sem_out)
        o1 = pltpu.async_copy(buf, disp_hbm.at[idx_v.at[2 * ch + 1]], sem_out)
        o0.wait()
        o1.wait()
        if ch + 1 < DNCH:
            nxt.wait()


def _dispatch(x, s0, s1):
    mesh = plsc.VectorSubcoreMesh(core_axis_name="c", subcore_axis_name="s")
    f = functools.partial(
        pl.kernel,
        mesh=mesh,
        compiler_params=pltpu.CompilerParams(needs_layout_passes=False),
        out_type=jax.ShapeDtypeStruct((NSLOT, D), jnp.float32),
        scratch_types=[
            pltpu.VMEM((2 * DNCH, DCH), jnp.int32),
            pltpu.VMEM((DCH, D), jnp.float32),
            pltpu.VMEM((DCH, D), jnp.float32),
            pltpu.SemaphoreType.DMA,
            pltpu.SemaphoreType.DMA,
        ],
    )(_dispatch_body)
    return f(x, s0, s1)


def _expert_body(disp_ref, we_ref, be_ref, y_ref):
    a = disp_ref[0]
    w = we_ref[0]
    y_ref[0] = (jnp.dot(a, w, preferred_element_type=jnp.float32)
                + be_ref[0])


def _expert_mm(disp, We, be3):
    return pl.pallas_call(
        _expert_body,
        grid=(E,),
        in_specs=[
            pl.BlockSpec((1, CAPP, D), lambda e: (e, 0, 0)),
            pl.BlockSpec((1, D, D), lambda e: (e, 0, 0)),
            pl.BlockSpec((1, 1, D), lambda e: (e, 0, 0)),
        ],
        out_specs=pl.BlockSpec((1, CAPP, D), lambda e: (e, 0, 0)),
        out_shape=jax.ShapeDtypeStruct((E, CAPP, D), jnp.float32),
    )(disp, We, be3)


def _combine_body(y_hbm, d0_hbm, d1_hbm, g0_hbm, g1_hbm, out_hbm,
                  d0_v, d1_v, g0_v, g1_v,
                  b0a_v, b1a_v, b0b_v, b1b_v, oba_v, obb_v,
                  sga, sgb, sw):
    wid = lax.axis_index("s") * NC + lax.axis_index("c")
    base = wid * TPT
    nch = TPT // CCH

    pltpu.sync_copy(d0_hbm.at[pl.ds(base, TPT)], d0_v)
    pltpu.sync_copy(d1_hbm.at[pl.ds(base, TPT)], d1_v)
    pltpu.sync_copy(g0_hbm.at[pl.ds(base, TPT)], g0_v)
    pltpu.sync_copy(g1_hbm.at[pl.ds(base, TPT)], g1_v)

    b0s = [b0a_v, b0b_v]
    b1s = [b1a_v, b1b_v]
    obs = [oba_v, obb_v]
    sgs = [sga, sgb]

    def gathers(ch, k):
        t0 = ch * CCH
        pltpu.async_copy(y_hbm.at[d0_v.at[pl.ds(t0, CCH)]], b0s[k], sgs[k])
        pltpu.async_copy(y_hbm.at[d1_v.at[pl.ds(t0, CCH)]], b1s[k], sgs[k])

    gathers(0, 0)
    for ch in range(nch):
        k = ch % 2
        if ch + 1 < nch:
            gathers(ch + 1, 1 - k)
        pltpu.make_async_copy(y_hbm.at[d0_v.at[pl.ds(0, CCH)]],
                              b0s[k], sgs[k]).wait()
        pltpu.make_async_copy(y_hbm.at[d1_v.at[pl.ds(0, CCH)]],
                              b1s[k], sgs[k]).wait()
        if ch >= 2:
            pltpu.make_async_copy(obs[k], out_hbm.at[pl.ds(0, CCH)],
                                  sw).wait()
        t0 = ch * CCH

        def tok_body(tt, _):
            for u in range(2):
                t = tt * 2 + u
                bcast = jnp.zeros((L,), jnp.int32) + (t0 + t)
                gt0 = plsc.load_gather(g0_v, [bcast])
                gt1 = plsc.load_gather(g1_v, [bcast])
                for j in range(D // L):
                    sl = pl.ds(j * L, L)
                    obs[k][t, sl] = (b0s[k][t, sl] * gt0
                                     + b1s[k][t, sl] * gt1)
            return 0
        lax.fori_loop(0, CCH // 2, tok_body, 0)
        pltpu.async_copy(obs[k], out_hbm.at[pl.ds(base + t0, CCH)], sw)
    pltpu.make_async_copy(obs[0], out_hbm.at[pl.ds(0, CCH)], sw).wait()
    pltpu.make_async_copy(obs[0], out_hbm.at[pl.ds(0, CCH)], sw).wait()


def _combine(y, d0, d1, g0, g1):
    mesh = plsc.VectorSubcoreMesh(core_axis_name="c", subcore_axis_name="s")
    f = functools.partial(
        pl.kernel,
        mesh=mesh,
        compiler_params=pltpu.CompilerParams(needs_layout_passes=False),
        out_type=jax.ShapeDtypeStruct((N, D), jnp.float32),
        scratch_types=[
            pltpu.VMEM((TPT,), jnp.int32),
            pltpu.VMEM((TPT,), jnp.int32),
            pltpu.VMEM((TPT,), jnp.float32),
            pltpu.VMEM((TPT,), jnp.float32),
            pltpu.VMEM((CCH, D), jnp.float32),
            pltpu.VMEM((CCH, D), jnp.float32),
            pltpu.VMEM((CCH, D), jnp.float32),
            pltpu.VMEM((CCH, D), jnp.float32),
            pltpu.VMEM((CCH, D), jnp.float32),
            pltpu.VMEM((CCH, D), jnp.float32),
            pltpu.SemaphoreType.DMA,
            pltpu.SemaphoreType.DMA,
            pltpu.SemaphoreType.DMA,
        ],
    )(_combine_body)
    return f(y, d0, d1, g0, g1)


def kernel(hidden_states, Wg, We, be):
    x = hidden_states
    rr = lax.broadcasted_iota(jnp.int32, (B, B), 0)
    cc = lax.broadcasted_iota(jnp.int32, (B, B), 1)
    tril = jnp.where(cc < rr, 1.0, 0.0).astype(jnp.float32)
    d0, d1, s0, s1, g0, g1 = _router(x, Wg, tril)
    d0 = d0.reshape(N)
    d1 = d1.reshape(N)
    s0 = s0.reshape(N)
    s1 = s1.reshape(N)
    g0 = g0.reshape(N)
    g1 = g1.reshape(N)

    disp = _dispatch(x, s0, s1)
    y = _expert_mm(disp.reshape(E, CAPP, D), We, be.reshape(E, 1, D))
    return _combine(y.reshape(NSLOT, D), d0, d1, g0, g1)


# speedup vs baseline: 1.2818x; 1.2818x over previous
import functools

import jax
import jax.numpy as jnp
from jax import lax
from jax.experimental import pallas as pl
from jax.experimental.pallas import tpu as pltpu
from jax.experimental.pallas import tpu_sc as plsc

E = 64
K = 2
D = 768
N = 4096
CAP = 160
CAPP = CAP + 8
NSLOT = E * CAPP
PAD_ROW = N

B = 512
NB = N // B

NC = 2
NS = 16
NW = NC * NS
L = 16

TPT = N // NW
CCH = 16


def _router_body(x_ref, wg_ref, tril_ref, d0_ref, d1_ref, s0_ref, s1_ref,
                 g0_ref, g1_ref, carry_ref):
    i = pl.program_id(0)

    @pl.when(i == 0)
    def _():
        carry_ref[...] = jnp.zeros_like(carry_ref)

    x = x_ref[...]
    wg = wg_ref[...]
    logits = jnp.dot(x, wg, preferred_element_type=jnp.float32)

    re = lax.broadcasted_iota(jnp.int32, (E, E), 0)
    ce = lax.broadcasted_iota(jnp.int32, (E, E), 1)
    ut = jnp.where(re < ce, 1.0, 0.0)
    iota_col = lax.broadcasted_iota(jnp.int32, (E, 1), 0).astype(jnp.float32)
    ones_col = jnp.full((E, 1), 1.0, jnp.float32)

    def first_max(v):
        m = jnp.max(v, axis=1, keepdims=True)
        eq = jnp.where(v == m, 1.0, 0.0)
        before = jnp.dot(eq, ut, preferred_element_type=jnp.float32)
        oh = eq * jnp.where(before == 0.0, 1.0, 0.0)
        idx = jnp.dot(oh, iota_col, preferred_element_type=jnp.float32)
        return m, oh, idx

    m0, oh0, idx0 = first_max(logits)
    masked = jnp.where(oh0 > 0.0, -jnp.inf, logits)
    m1, oh1, idx1 = first_max(masked)

    t = jnp.exp(m1 - m0)
    g0 = 1.0 / (1.0 + t)
    g1 = 1.0 - g0

    ohsum = oh0 + oh1
    cnt_before = (jnp.dot(tril_ref[...], ohsum,
                          preferred_element_type=jnp.float32)
                  + carry_ref[...])
    carry_ref[...] = carry_ref[...] + jnp.sum(ohsum, axis=0, keepdims=True)

    pos0 = jnp.dot(oh0 * cnt_before, ones_col,
                   preferred_element_type=jnp.float32)
    pos1 = jnp.dot(oh1 * cnt_before, ones_col,
                   preferred_element_type=jnp.float32)
    keep0 = pos0 < CAP
    keep1 = pos1 < CAP
    base0 = idx0 * CAPP
    base1 = idx1 * CAPP
    d0 = base0 + jnp.minimum(pos0, CAP - 1.0)
    d1 = base1 + jnp.minimum(pos1, CAP - 1.0)

    d0_ref[...] = d0.astype(jnp.int32)
    d1_ref[...] = d1.astype(jnp.int32)
    s0_ref[...] = (base0 + jnp.minimum(pos0, float(CAP))).astype(jnp.int32)
    s1_ref[...] = (base1 + jnp.minimum(pos1, float(CAP))).astype(jnp.int32)
    g0_ref[...] = g0 * keep0.astype(jnp.float32)
    g1_ref[...] = g1 * keep1.astype(jnp.float32)


def _router(x, Wg, tril):
    blk = pl.BlockSpec((B, 1), lambda i: (i, 0))
    iod = jax.ShapeDtypeStruct((N, 1), jnp.int32)
    fod = jax.ShapeDtypeStruct((N, 1), jnp.float32)
    return pl.pallas_call(
        _router_body,
        grid=(NB,),
        in_specs=[
            pl.BlockSpec((B, D), lambda i: (i, 0)),
            pl.BlockSpec((D, E), lambda i: (0, 0)),
            pl.BlockSpec((B, B), lambda i: (0, 0)),
        ],
        out_specs=[blk] * 6,
        out_shape=[iod, iod, iod, iod, fod, fod],
        scratch_shapes=[pltpu.VMEM((1, E), jnp.float32)],
    )(x, Wg, tril)


DCH = 64
DNCH = (N // NW) // DCH


def _dispatch_body(x_hbm, s0_hbm, s1_hbm, disp_hbm,
                   idx_v, xb0_v, xb1_v, sem_in, sem_out):
    wid = lax.axis_index("s") * NC + lax.axis_index("c")
    base = wid * (N // NW)

    for ch in range(DNCH):
        t0 = base + ch * DCH
        pltpu.sync_copy(s0_hbm.at[pl.ds(t0, DCH)], idx_v.at[2 * ch])
        pltpu.sync_copy(s1_hbm.at[pl.ds(t0, DCH)], idx_v.at[2 * ch + 1])

    bufs = [xb0_v, xb1_v]
    cp = pltpu.async_copy(x_hbm.at[pl.ds(base, DCH)], bufs[0], sem_in)
    cp.wait()
    for ch in range(DNCH):
        buf = bufs[ch % 2]
        if ch + 1 < DNCH:
            nxt = pltpu.async_copy(
                x_hbm.at[pl.ds(base + (ch + 1) * DCH, DCH)],
                bufs[(ch + 1) % 2], sem_in)
        o0 = pltpu.async_copy(buf, disp_hbm.at[idx_v.at[2 * ch]], sem_out)
        o1 = pltpu.async_copy(buf, disp_hbm.at[idx_v.at[2 * ch + 1]], sem_out)
        o0.wait()
        o1.wait()
        if ch + 1 < DNCH:
            nxt.wait()


def _dispatch(x, s0, s1):
    mesh = plsc.VectorSubcoreMesh(core_axis_name="c", subcore_axis_name="s")
    f = functools.partial(
        pl.kernel,
        mesh=mesh,
        compiler_params=pltpu.CompilerParams(needs_layout_passes=False),
        out_type=jax.ShapeDtypeStruct((NSLOT, D), jnp.float32),
        scratch_types=[
            pltpu.VMEM((2 * DNCH, DCH), jnp.int32),
            pltpu.VMEM((DCH, D), jnp.float32),
            pltpu.VMEM((DCH, D), jnp.float32),
            pltpu.SemaphoreType.DMA,
            pltpu.SemaphoreType.DMA,
        ],
    )(_dispatch_body)
    return f(x, s0, s1)


def _expert_body(disp_ref, we_ref, be_ref, y_ref):
    a = disp_ref[0]
    w = we_ref[0]
    y_ref[0] = (jnp.dot(a, w, preferred_element_type=jnp.float32)
                + be_ref[0])


def _expert_mm(disp, We, be3):
    return pl.pallas_call(
        _expert_body,
        grid=(E,),
        in_specs=[
            pl.BlockSpec((1, CAPP, D), lambda e: (e, 0, 0)),
            pl.BlockSpec((1, D, D), lambda e: (e, 0, 0)),
            pl.BlockSpec((1, 1, D), lambda e: (e, 0, 0)),
        ],
        out_specs=pl.BlockSpec((1, CAPP, D), lambda e: (e, 0, 0)),
        out_shape=jax.ShapeDtypeStruct((E, CAPP, D), jnp.float32),
    )(disp, We, be3)


def _combine_body(y_hbm, d0_hbm, d1_hbm, g0_hbm, g1_hbm, out_hbm,
                  d0_v, d1_v, g0_v, g1_v,
                  b0a_v, b1a_v, b0b_v, b1b_v, oba_v, obb_v,
                  sga, sgb, sw):
    wid = lax.axis_index("s") * NC + lax.axis_index("c")
    base = wid * TPT
    nch = TPT // CCH

    pltpu.sync_copy(d0_hbm.at[pl.ds(base, TPT)], d0_v)
    pltpu.sync_copy(d1_hbm.at[pl.ds(base, TPT)], d1_v)
    pltpu.sync_copy(g0_hbm.at[pl.ds(base, TPT)], g0_v)
    pltpu.sync_copy(g1_hbm.at[pl.ds(base, TPT)], g1_v)

    b0s = [b0a_v, b0b_v]
    b1s = [b1a_v, b1b_v]
    obs = [oba_v, obb_v]
    sgs = [sga, sgb]

    def gathers(ch, k):
        t0 = ch * CCH
        pltpu.async_copy(y_hbm.at[d0_v.at[pl.ds(t0, CCH)]], b0s[k], sgs[k])
        pltpu.async_copy(y_hbm.at[d1_v.at[pl.ds(t0, CCH)]], b1s[k], sgs[k])

    gathers(0, 0)
    for ch in range(nch):
        k = ch % 2
        if ch + 1 < nch:
            gathers(ch + 1, 1 - k)
        pltpu.make_async_copy(y_hbm.at[d0_v.at[pl.ds(0, CCH)]],
                              b0s[k], sgs[k]).wait()
        pltpu.make_async_copy(y_hbm.at[d1_v.at[pl.ds(0, CCH)]],
                              b1s[k], sgs[k]).wait()
        if ch >= 2:
            pltpu.make_async_copy(obs[k], out_hbm.at[pl.ds(0, CCH)],
                                  sw).wait()
        t0 = ch * CCH

        def tok_body(tt, _):
            for u in range(2):
                t = tt * 2 + u
                bcast = jnp.zeros((L,), jnp.int32) + (t0 + t)
                gt0 = plsc.load_gather(g0_v, [bcast])
                gt1 = plsc.load_gather(g1_v, [bcast])
                for j in range(D // L):
                    sl = pl.ds(j * L, L)
                    obs[k][t, sl] = (b0s[k][t, sl] * gt0
                                     + b1s[k][t, sl] * gt1)
            return 0
        lax.fori_loop(0, CCH // 2, tok_body, 0)
        pltpu.async_copy(obs[k], out_hbm.at[pl.ds(base + t0, CCH)], sw)
    pltpu.make_async_copy(obs[0], out_hbm.at[pl.ds(0, CCH)], sw).wait()
    pltpu.make_async_copy(obs[0], out_hbm.at[pl.ds(0, CCH)], sw).wait()


def _combine(y, d0, d1, g0, g1):
    mesh = plsc.VectorSubcoreMesh(core_axis_name="c", subcore_axis_name="s")
    f = functools.partial(
        pl.kernel,
        mesh=mesh,
        compiler_params=pltpu.CompilerParams(needs_layout_passes=False),
        out_type=jax.ShapeDtypeStruct((N, D), jnp.float32),
        scratch_types=[
            pltpu.VMEM((TPT,), jnp.int32),
            pltpu.VMEM((TPT,), jnp.int32),
            pltpu.VMEM((TPT,), jnp.float32),
            pltpu.VMEM((TPT,), jnp.float32),
            pltpu.VMEM((CCH, D), jnp.float32),
            pltpu.VMEM((CCH, D), jnp.float32),
            pltpu.VMEM((CCH, D), jnp.float32),
            pltpu.VMEM((CCH, D), jnp.float32),
            pltpu.VMEM((CCH, D), jnp.float32),
            pltpu.VMEM((CCH, D), jnp.float32),
            pltpu.SemaphoreType.DMA,
            pltpu.SemaphoreType.DMA,
            pltpu.SemaphoreType.DMA,
        ],
    )(_combine_body)
    return f(y, d0, d1, g0, g1)


def kernel(hidden_states, Wg, We, be):
    x = hidden_states
    rr = lax.broadcasted_iota(jnp.int32, (B, B), 0)
    cc = lax.broadcasted_iota(jnp.int32, (B, B), 1)
    tril = jnp.where(cc < rr, 1.0, 0.0).astype(jnp.float32)
    d0, d1, s0, s1, g0, g1 = _router(x, Wg, tril)
    d0 = d0.reshape(N)
    d1 = d1.reshape(N)
    s0 = s0.reshape(N)
    s1 = s1.reshape(N)
    g0 = g0.reshape(N)
    g1 = g1.reshape(N)

    disp = _dispatch(x, s0, s1)
    y = _expert_mm(disp.reshape(E, CAPP, D), We, be.reshape(E, 1, D))
    return _combine(y.reshape(NSLOT, D), d0, d1, g0, g1)


# speedup vs baseline: 1.2920x; 1.0079x over previous
import functools

import jax
import jax.numpy as jnp
from jax import lax
from jax.experimental import pallas as pl
from jax.experimental.pallas import tpu as pltpu
from jax.experimental.pallas import tpu_sc as plsc

E = 64
K = 2
D = 768
N = 4096
CAP = 160
CAPP = CAP + 8
NSLOT = E * CAPP
PAD_ROW = N

B = 512
NB = N // B

NC = 2
NS = 16
NW = NC * NS
L = 16

TPT = N // NW
CCH = 16


def _router_body(x_ref, wg_ref, d0_ref, d1_ref, s0_ref, s1_ref,
                 g0_ref, g1_ref, carry_ref, tril_ref):
    i = pl.program_id(0)

    @pl.when(i == 0)
    def _():
        carry_ref[...] = jnp.zeros_like(carry_ref)
        rr = lax.broadcasted_iota(jnp.int32, (B, B), 0)
        cc = lax.broadcasted_iota(jnp.int32, (B, B), 1)
        tril_ref[...] = jnp.where(cc < rr, 1.0, 0.0)

    x = x_ref[...]
    wg = wg_ref[...]
    logits = jnp.dot(x, wg, preferred_element_type=jnp.float32)

    re = lax.broadcasted_iota(jnp.int32, (E, E), 0)
    ce = lax.broadcasted_iota(jnp.int32, (E, E), 1)
    ut = jnp.where(re < ce, 1.0, 0.0)
    iota_col = lax.broadcasted_iota(jnp.int32, (E, 1), 0).astype(jnp.float32)
    ones_col = jnp.full((E, 1), 1.0, jnp.float32)

    def first_max(v):
        m = jnp.max(v, axis=1, keepdims=True)
        eq = jnp.where(v == m, 1.0, 0.0)
        before = jnp.dot(eq, ut, preferred_element_type=jnp.float32)
        oh = eq * jnp.where(before == 0.0, 1.0, 0.0)
        idx = jnp.dot(oh, iota_col, preferred_element_type=jnp.float32)
        return m, oh, idx

    m0, oh0, idx0 = first_max(logits)
    masked = jnp.where(oh0 > 0.0, -jnp.inf, logits)
    m1, oh1, idx1 = first_max(masked)

    t = jnp.exp(m1 - m0)
    g0 = 1.0 / (1.0 + t)
    g1 = 1.0 - g0

    ohsum = oh0 + oh1
    cnt_before = (jnp.dot(tril_ref[...], ohsum,
                          preferred_element_type=jnp.float32)
                  + carry_ref[...])
    carry_ref[...] = carry_ref[...] + jnp.sum(ohsum, axis=0, keepdims=True)

    pos0 = jnp.dot(oh0 * cnt_before, ones_col,
                   preferred_element_type=jnp.float32)
    pos1 = jnp.dot(oh1 * cnt_before, ones_col,
                   preferred_element_type=jnp.float32)
    keep0 = pos0 < CAP
    keep1 = pos1 < CAP
    base0 = idx0 * CAPP
    base1 = idx1 * CAPP
    d0 = base0 + jnp.minimum(pos0, CAP - 1.0)
    d1 = base1 + jnp.minimum(pos1, CAP - 1.0)

    d0_ref[...] = d0.astype(jnp.int32)
    d1_ref[...] = d1.astype(jnp.int32)
    s0_ref[...] = (base0 + jnp.minimum(pos0, float(CAP))).astype(jnp.int32)
    s1_ref[...] = (base1 + jnp.minimum(pos1, float(CAP))).astype(jnp.int32)
    g0_ref[...] = g0 * keep0.astype(jnp.float32)
    g1_ref[...] = g1 * keep1.astype(jnp.float32)


def _router(x, Wg):
    blk = pl.BlockSpec((B, 1), lambda i: (i, 0))
    iod = jax.ShapeDtypeStruct((N, 1), jnp.int32)
    fod = jax.ShapeDtypeStruct((N, 1), jnp.float32)
    return pl.pallas_call(
        _router_body,
        grid=(NB,),
        in_specs=[
            pl.BlockSpec((B, D), lambda i: (i, 0)),
            pl.BlockSpec((D, E), lambda i: (0, 0)),
        ],
        out_specs=[blk] * 6,
        out_shape=[iod, iod, iod, iod, fod, fod],
        scratch_shapes=[pltpu.VMEM((1, E), jnp.float32),
                        pltpu.VMEM((B, B), jnp.float32)],
    )(x, Wg)


DCH = 64
DNCH = (N // NW) // DCH


def _dispatch_body(x_hbm, s0_hbm, s1_hbm, disp_hbm,
                   idx_v, xb0_v, xb1_v, sem_in, sem_out):
    wid = lax.axis_index("s") * NC + lax.axis_index("c")
    base = wid * (N // NW)

    for ch in range(DNCH):
        t0 = base + ch * DCH
        pltpu.sync_copy(s0_hbm.at[pl.ds(t0, DCH)], idx_v.at[2 * ch])
        pltpu.sync_copy(s1_hbm.at[pl.ds(t0, DCH)], idx_v.at[2 * ch + 1])

    bufs = [xb0_v, xb1_v]
    cp = pltpu.async_copy(x_hbm.at[pl.ds(base, DCH)], bufs[0], sem_in)
    cp.wait()
    for ch in range(DNCH):
        buf = bufs[ch % 2]
        if ch + 1 < DNCH:
            nxt = pltpu.async_copy(
                x_hbm.at[pl.ds(base + (ch + 1) * DCH, DCH)],
                bufs[(ch + 1) % 2], sem_in)
        o0 = pltpu.async_copy(buf, disp_hbm.at[idx_v.at[2 * ch]], sem_out)
        o1 = pltpu.async_copy(buf, disp_hbm.at[idx_v.at[2 * ch + 1]], sem_out)
        o0.wait()
        o1.wait()
        if ch + 1 < DNCH:
            nxt.wait()


def _dispatch(x, s0, s1):
    mesh = plsc.VectorSubcoreMesh(core_axis_name="c", subcore_axis_name="s")
    f = functools.partial(
        pl.kernel,
        mesh=mesh,
        compiler_params=pltpu.CompilerParams(needs_layout_passes=False),
        out_type=jax.ShapeDtypeStruct((NSLOT, D), jnp.float32),
        scratch_types=[
            pltpu.VMEM((2 * DNCH, DCH), jnp.int32),
            pltpu.VMEM((DCH, D), jnp.float32),
            pltpu.VMEM((DCH, D), jnp.float32),
            pltpu.SemaphoreType.DMA,
            pltpu.SemaphoreType.DMA,
        ],
    )(_dispatch_body)
    return f(x, s0, s1)


def _expert_body(disp_ref, we_ref, be_ref, y_ref):
    a = disp_ref[0]
    w = we_ref[0]
    y_ref[0] = (jnp.dot(a, w, preferred_element_type=jnp.float32)
                + be_ref[0])


def _expert_mm(disp, We, be3):
    return pl.pallas_call(
        _expert_body,
        grid=(E,),
        in_specs=[
            pl.BlockSpec((1, CAPP, D), lambda e: (e, 0, 0)),
            pl.BlockSpec((1, D, D), lambda e: (e, 0, 0)),
            pl.BlockSpec((1, 1, D), lambda e: (e, 0, 0)),
        ],
        out_specs=pl.BlockSpec((1, CAPP, D), lambda e: (e, 0, 0)),
        out_shape=jax.ShapeDtypeStruct((E, CAPP, D), jnp.float32),
    )(disp, We, be3)


def _combine_body(y_hbm, d0_hbm, d1_hbm, g0_hbm, g1_hbm, out_hbm,
                  d0_v, d1_v, g0_v, g1_v,
                  b0a_v, b1a_v, b0b_v, b1b_v, oba_v, obb_v,
                  sga, sgb, sw):
    wid = lax.axis_index("s") * NC + lax.axis_index("c")
    base = wid * TPT
    nch = TPT // CCH

    pltpu.sync_copy(d0_hbm.at[pl.ds(base, TPT)], d0_v)
    pltpu.sync_copy(d1_hbm.at[pl.ds(base, TPT)], d1_v)
    pltpu.sync_copy(g0_hbm.at[pl.ds(base, TPT)], g0_v)
    pltpu.sync_copy(g1_hbm.at[pl.ds(base, TPT)], g1_v)

    b0s = [b0a_v, b0b_v]
    b1s = [b1a_v, b1b_v]
    obs = [oba_v, obb_v]
    sgs = [sga, sgb]

    def gathers(ch, k):
        t0 = ch * CCH
        pltpu.async_copy(y_hbm.at[d0_v.at[pl.ds(t0, CCH)]], b0s[k], sgs[k])
        pltpu.async_copy(y_hbm.at[d1_v.at[pl.ds(t0, CCH)]], b1s[k], sgs[k])

    gathers(0, 0)
    for ch in range(nch):
        k = ch % 2
        if ch + 1 < nch:
            gathers(ch + 1, 1 - k)
        pltpu.make_async_copy(y_hbm.at[d0_v.at[pl.ds(0, CCH)]],
                              b0s[k], sgs[k]).wait()
        pltpu.make_async_copy(y_hbm.at[d1_v.at[pl.ds(0, CCH)]],
                              b1s[k], sgs[k]).wait()
        if ch >= 2:
            pltpu.make_async_copy(obs[k], out_hbm.at[pl.ds(0, CCH)],
                                  sw).wait()
        t0 = ch * CCH

        def tok_body(tt, _):
            for u in range(2):
                t = tt * 2 + u
                bcast = jnp.zeros((L,), jnp.int32) + (t0 + t)
                gt0 = plsc.load_gather(g0_v, [bcast])
                gt1 = plsc.load_gather(g1_v, [bcast])
                for j in range(D // L):
                    sl = pl.ds(j * L, L)
                    obs[k][t, sl] = (b0s[k][t, sl] * gt0
                                     + b1s[k][t, sl] * gt1)
            return 0
        lax.fori_loop(0, CCH // 2, tok_body, 0)
        pltpu.async_copy(obs[k], out_hbm.at[pl.ds(base + t0, CCH)], sw)
    pltpu.make_async_copy(obs[0], out_hbm.at[pl.ds(0, CCH)], sw).wait()
    pltpu.make_async_copy(obs[0], out_hbm.at[pl.ds(0, CCH)], sw).wait()


def _combine(y, d0, d1, g0, g1):
    mesh = plsc.VectorSubcoreMesh(core_axis_name="c", subcore_axis_name="s")
    f = functools.partial(
        pl.kernel,
        mesh=mesh,
        compiler_params=pltpu.CompilerParams(needs_layout_passes=False),
        out_type=jax.ShapeDtypeStruct((N, D), jnp.float32),
        scratch_types=[
            pltpu.VMEM((TPT,), jnp.int32),
            pltpu.VMEM((TPT,), jnp.int32),
            pltpu.VMEM((TPT,), jnp.float32),
            pltpu.VMEM((TPT,), jnp.float32),
            pltpu.VMEM((CCH, D), jnp.float32),
            pltpu.VMEM((CCH, D), jnp.float32),
            pltpu.VMEM((CCH, D), jnp.float32),
            pltpu.VMEM((CCH, D), jnp.float32),
            pltpu.VMEM((CCH, D), jnp.float32),
            pltpu.VMEM((CCH, D), jnp.float32),
            pltpu.SemaphoreType.DMA,
            pltpu.SemaphoreType.DMA,
            pltpu.SemaphoreType.DMA,
        ],
    )(_combine_body)
    return f(y, d0, d1, g0, g1)


def kernel(hidden_states, Wg, We, be):
    x = hidden_states
    d0, d1, s0, s1, g0, g1 = _router(x, Wg)
    d0 = d0.reshape(N)
    d1 = d1.reshape(N)
    s0 = s0.reshape(N)
    s1 = s1.reshape(N)
    g0 = g0.reshape(N)
    g1 = g1.reshape(N)

    disp = _dispatch(x, s0, s1)
    y = _expert_mm(disp.reshape(E, CAPP, D), We, be.reshape(E, 1, D))
    return _combine(y.reshape(NSLOT, D), d0, d1, g0, g1)


# speedup vs baseline: 1.3206x; 1.0222x over previous
import functools

import jax
import jax.numpy as jnp
from jax import lax
from jax.experimental import pallas as pl
from jax.experimental.pallas import tpu as pltpu
from jax.experimental.pallas import tpu_sc as plsc

E = 64
K = 2
D = 768
N = 4096
CAP = 160
CAPP = CAP + 8
NSLOT = E * CAPP
PAD_ROW = N

B = 512
NB = N // B

NC = 2
NS = 16
NW = NC * NS
L = 16

TPT = N // NW
CCH = 16


def _router_body(x_ref, wg_ref, d0_ref, d1_ref, s0_ref, s1_ref,
                 g0_ref, g1_ref, carry_ref, tril_ref):
    i = pl.program_id(0)

    @pl.when(i == 0)
    def _():
        carry_ref[...] = jnp.zeros_like(carry_ref)
        rr = lax.broadcasted_iota(jnp.int32, (B, B), 0)
        cc = lax.broadcasted_iota(jnp.int32, (B, B), 1)
        tril_ref[...] = jnp.where(cc < rr, 1.0, 0.0)

    x = x_ref[...]
    wg = wg_ref[...]
    logits = jnp.dot(x, wg, preferred_element_type=jnp.float32)

    re = lax.broadcasted_iota(jnp.int32, (E, E), 0)
    ce = lax.broadcasted_iota(jnp.int32, (E, E), 1)
    ut = jnp.where(re < ce, 1.0, 0.0)
    iota_col = lax.broadcasted_iota(jnp.int32, (E, 1), 0).astype(jnp.float32)
    ones_col = jnp.full((E, 1), 1.0, jnp.float32)

    def first_max(v):
        m = jnp.max(v, axis=1, keepdims=True)
        eq = jnp.where(v == m, 1.0, 0.0)
        before = jnp.dot(eq, ut, preferred_element_type=jnp.float32)
        oh = eq * jnp.where(before == 0.0, 1.0, 0.0)
        idx = jnp.dot(oh, iota_col, preferred_element_type=jnp.float32)
        return m, oh, idx

    m0, oh0, idx0 = first_max(logits)
    masked = jnp.where(oh0 > 0.0, -jnp.inf, logits)
    m1, oh1, idx1 = first_max(masked)

    t = jnp.exp(m1 - m0)
    g0 = 1.0 / (1.0 + t)
    g1 = 1.0 - g0

    ohsum = oh0 + oh1
    cnt_before = (jnp.dot(tril_ref[...], ohsum,
                          preferred_element_type=jnp.float32)
                  + carry_ref[...])
    carry_ref[...] = carry_ref[...] + jnp.sum(ohsum, axis=0, keepdims=True)

    pos0 = jnp.dot(oh0 * cnt_before, ones_col,
                   preferred_element_type=jnp.float32)
    pos1 = jnp.dot(oh1 * cnt_before, ones_col,
                   preferred_element_type=jnp.float32)
    keep0 = pos0 < CAP
    keep1 = pos1 < CAP
    base0 = idx0 * CAPP
    base1 = idx1 * CAPP
    d0 = base0 + jnp.minimum(pos0, CAP - 1.0)
    d1 = base1 + jnp.minimum(pos1, CAP - 1.0)

    d0_ref[...] = d0.astype(jnp.int32)
    d1_ref[...] = d1.astype(jnp.int32)
    s0_ref[...] = (base0 + jnp.minimum(pos0, float(CAP))).astype(jnp.int32)
    s1_ref[...] = (base1 + jnp.minimum(pos1, float(CAP))).astype(jnp.int32)
    g0_ref[...] = g0 * keep0.astype(jnp.float32)
    g1_ref[...] = g1 * keep1.astype(jnp.float32)


def _router(x, Wg):
    blk = pl.BlockSpec((B, 1), lambda i: (i, 0))
    iod = jax.ShapeDtypeStruct((N, 1), jnp.int32)
    fod = jax.ShapeDtypeStruct((N, 1), jnp.float32)
    return pl.pallas_call(
        _router_body,
        grid=(NB,),
        in_specs=[
            pl.BlockSpec((B, D), lambda i: (i, 0)),
            pl.BlockSpec((D, E), lambda i: (0, 0)),
        ],
        out_specs=[blk] * 6,
        out_shape=[iod, iod, iod, iod, fod, fod],
        scratch_shapes=[pltpu.VMEM((1, E), jnp.float32),
                        pltpu.VMEM((B, B), jnp.float32)],
    )(x, Wg)


DCH = 64
DNCH = (N // NW) // DCH


def _dispatch_body(x_hbm, s0_hbm, s1_hbm, disp_hbm,
                   idx_v, xb0_v, xb1_v, sem_in, sem_out):
    wid = lax.axis_index("s") * NC + lax.axis_index("c")
    base = wid * (N // NW)

    pcopies = []
    for ch in range(DNCH):
        t0 = base + ch * DCH
        pcopies.append(pltpu.async_copy(
            s0_hbm.at[pl.ds(t0, DCH)], idx_v.at[2 * ch], sem_in))
        pcopies.append(pltpu.async_copy(
            s1_hbm.at[pl.ds(t0, DCH)], idx_v.at[2 * ch + 1], sem_in))
    for cp in pcopies:
        cp.wait()

    bufs = [xb0_v, xb1_v]
    cp = pltpu.async_copy(x_hbm.at[pl.ds(base, DCH)], bufs[0], sem_in)
    cp.wait()
    for ch in range(DNCH):
        buf = bufs[ch % 2]
        if ch + 1 < DNCH:
            nxt = pltpu.async_copy(
                x_hbm.at[pl.ds(base + (ch + 1) * DCH, DCH)],
                bufs[(ch + 1) % 2], sem_in)
        o0 = pltpu.async_copy(buf, disp_hbm.at[idx_v.at[2 * ch]], sem_out)
        o1 = pltpu.async_copy(buf, disp_hbm.at[idx_v.at[2 * ch + 1]], sem_out)
        o0.wait()
        o1.wait()
        if ch + 1 < DNCH:
            nxt.wait()


def _dispatch(x, s0, s1):
    mesh = plsc.VectorSubcoreMesh(core_axis_name="c", subcore_axis_name="s")
    f = functools.partial(
        pl.kernel,
        mesh=mesh,
        compiler_params=pltpu.CompilerParams(needs_layout_passes=False),
        out_type=jax.ShapeDtypeStruct((NSLOT, D), jnp.float32),
        scratch_types=[
            pltpu.VMEM((2 * DNCH, DCH), jnp.int32),
            pltpu.VMEM((DCH, D), jnp.float32),
            pltpu.VMEM((DCH, D), jnp.float32),
            pltpu.SemaphoreType.DMA,
            pltpu.SemaphoreType.DMA,
        ],
    )(_dispatch_body)
    return f(x, s0, s1)


def _expert_body(disp_ref, we_ref, be_ref, y_ref):
    a = disp_ref[0]
    w = we_ref[0]
    y_ref[0] = (jnp.dot(a, w, preferred_element_type=jnp.float32)
                + be_ref[0])


def _expert_mm(disp, We, be3):
    return pl.pallas_call(
        _expert_body,
        grid=(E,),
        in_specs=[
            pl.BlockSpec((1, CAPP, D), lambda e: (e, 0, 0)),
            pl.BlockSpec((1, D, D), lambda e: (e, 0, 0)),
            pl.BlockSpec((1, 1, D), lambda e: (e, 0, 0)),
        ],
        out_specs=pl.BlockSpec((1, CAPP, D), lambda e: (e, 0, 0)),
        out_shape=jax.ShapeDtypeStruct((E, CAPP, D), jnp.float32),
    )(disp, We, be3)


def _combine_body(y_hbm, d0_hbm, d1_hbm, g0_hbm, g1_hbm, out_hbm,
                  d0_v, d1_v, g0_v, g1_v,
                  b0a_v, b1a_v, b0b_v, b1b_v, b0c_v, b1c_v, oba_v, obb_v,
                  sga, sgb, sgc, sw):
    wid = lax.axis_index("s") * NC + lax.axis_index("c")
    base = wid * TPT
    nch = TPT // CCH

    cps = [pltpu.async_copy(d0_hbm.at[pl.ds(base, TPT)], d0_v, sw),
           pltpu.async_copy(d1_hbm.at[pl.ds(base, TPT)], d1_v, sw),
           pltpu.async_copy(g0_hbm.at[pl.ds(base, TPT)], g0_v, sw),
           pltpu.async_copy(g1_hbm.at[pl.ds(base, TPT)], g1_v, sw)]
    for cp in cps:
        cp.wait()

    b0s = [b0a_v, b0b_v, b0c_v]
    b1s = [b1a_v, b1b_v, b1c_v]
    obs = [oba_v, obb_v]
    sgs = [sga, sgb, sgc]

    def gathers(ch, k):
        t0 = ch * CCH
        pltpu.async_copy(y_hbm.at[d0_v.at[pl.ds(t0, CCH)]], b0s[k], sgs[k])
        pltpu.async_copy(y_hbm.at[d1_v.at[pl.ds(t0, CCH)]], b1s[k], sgs[k])

    gathers(0, 0)
    gathers(1, 1)
    for ch in range(nch):
        k = ch % 3
        ko = ch % 2
        if ch + 2 < nch:
            gathers(ch + 2, (ch + 2) % 3)
        pltpu.make_async_copy(y_hbm.at[d0_v.at[pl.ds(0, CCH)]],
                              b0s[k], sgs[k]).wait()
        pltpu.make_async_copy(y_hbm.at[d1_v.at[pl.ds(0, CCH)]],
                              b1s[k], sgs[k]).wait()
        if ch >= 2:
            pltpu.make_async_copy(obs[ko], out_hbm.at[pl.ds(0, CCH)],
                                  sw).wait()
        t0 = ch * CCH

        def tok_body(tt, _):
            for u in range(2):
                t = tt * 2 + u
                bcast = jnp.zeros((L,), jnp.int32) + (t0 + t)
                gt0 = plsc.load_gather(g0_v, [bcast])
                gt1 = plsc.load_gather(g1_v, [bcast])
                for j in range(D // L):
                    sl = pl.ds(j * L, L)
                    obs[ko][t, sl] = (b0s[k][t, sl] * gt0
                                      + b1s[k][t, sl] * gt1)
            return 0
        lax.fori_loop(0, CCH // 2, tok_body, 0)
        pltpu.async_copy(obs[ko], out_hbm.at[pl.ds(base + t0, CCH)], sw)
    pltpu.make_async_copy(obs[0], out_hbm.at[pl.ds(0, CCH)], sw).wait()
    pltpu.make_async_copy(obs[0], out_hbm.at[pl.ds(0, CCH)], sw).wait()


def _combine(y, d0, d1, g0, g1):
    mesh = plsc.VectorSubcoreMesh(core_axis_name="c", subcore_axis_name="s")
    f = functools.partial(
        pl.kernel,
        mesh=mesh,
        compiler_params=pltpu.CompilerParams(needs_layout_passes=False),
        out_type=jax.ShapeDtypeStruct((N, D), jnp.float32),
        scratch_types=[
            pltpu.VMEM((TPT,), jnp.int32),
            pltpu.VMEM((TPT,), jnp.int32),
            pltpu.VMEM((TPT,), jnp.float32),
            pltpu.VMEM((TPT,), jnp.float32),
            pltpu.VMEM((CCH, D), jnp.float32),
            pltpu.VMEM((CCH, D), jnp.float32),
            pltpu.VMEM((CCH, D), jnp.float32),
            pltpu.VMEM((CCH, D), jnp.float32),
            pltpu.VMEM((CCH, D), jnp.float32),
            pltpu.VMEM((CCH, D), jnp.float32),
            pltpu.VMEM((CCH, D), jnp.float32),
            pltpu.VMEM((CCH, D), jnp.float32),
            pltpu.SemaphoreType.DMA,
            pltpu.SemaphoreType.DMA,
            pltpu.SemaphoreType.DMA,
            pltpu.SemaphoreType.DMA,
        ],
    )(_combine_body)
    return f(y, d0, d1, g0, g1)


def kernel(hidden_states, Wg, We, be):
    x = hidden_states
    d0, d1, s0, s1, g0, g1 = _router(x, Wg)
    d0 = d0.reshape(N)
    d1 = d1.reshape(N)
    s0 = s0.reshape(N)
    s1 = s1.reshape(N)
    g0 = g0.reshape(N)
    g1 = g1.reshape(N)

    disp = _dispatch(x, s0, s1)
    y = _expert_mm(disp.reshape(E, CAPP, D), We, be.reshape(E, 1, D))
    return _combine(y.reshape(NSLOT, D), d0, d1, g0, g1)


# speedup vs baseline: 1.3234x; 1.0021x over previous
import functools

import jax
import jax.numpy as jnp
from jax import lax
from jax.experimental import pallas as pl
from jax.experimental.pallas import tpu as pltpu
from jax.experimental.pallas import tpu_sc as plsc

E = 64
K = 2
D = 768
N = 4096
CAP = 160
CAPP = CAP + 8
NSLOT = E * CAPP
PAD_ROW = N

B = 512
NB = N // B

NC = 2
NS = 16
NW = NC * NS
L = 16

TPT = N // NW
CCH = 16


def _router_body(x_ref, wg_ref, d0_ref, d1_ref, s0_ref, s1_ref,
                 g0_ref, g1_ref, carry_ref, tril_ref):
    i = pl.program_id(0)

    @pl.when(i == 0)
    def _():
        carry_ref[...] = jnp.zeros_like(carry_ref)
        rr = lax.broadcasted_iota(jnp.int32, (B, B), 0)
        cc = lax.broadcasted_iota(jnp.int32, (B, B), 1)
        tril_ref[...] = jnp.where(cc < rr, 1.0, 0.0)

    x = x_ref[...]
    wg = wg_ref[...]
    logits = jnp.dot(x, wg, preferred_element_type=jnp.float32)

    re = lax.broadcasted_iota(jnp.int32, (E, E), 0)
    ce = lax.broadcasted_iota(jnp.int32, (E, E), 1)
    ut = jnp.where(re < ce, 1.0, 0.0)
    iota_col = lax.broadcasted_iota(jnp.int32, (E, 1), 0).astype(jnp.float32)
    ones_col = jnp.full((E, 1), 1.0, jnp.float32)

    def first_max(v):
        m = jnp.max(v, axis=1, keepdims=True)
        eq = jnp.where(v == m, 1.0, 0.0)
        before = jnp.dot(eq, ut, preferred_element_type=jnp.float32)
        oh = eq * jnp.where(before == 0.0, 1.0, 0.0)
        idx = jnp.dot(oh, iota_col, preferred_element_type=jnp.float32)
        return m, oh, idx

    m0, oh0, idx0 = first_max(logits)
    masked = jnp.where(oh0 > 0.0, -jnp.inf, logits)
    m1, oh1, idx1 = first_max(masked)

    t = jnp.exp(m1 - m0)
    g0 = 1.0 / (1.0 + t)
    g1 = 1.0 - g0

    ohsum = oh0 + oh1
    cnt_before = (jnp.dot(tril_ref[...], ohsum,
                          preferred_element_type=jnp.float32)
                  + carry_ref[...])
    carry_ref[...] = carry_ref[...] + jnp.sum(ohsum, axis=0, keepdims=True)

    pos0 = jnp.dot(oh0 * cnt_before, ones_col,
                   preferred_element_type=jnp.float32)
    pos1 = jnp.dot(oh1 * cnt_before, ones_col,
                   preferred_element_type=jnp.float32)
    keep0 = pos0 < CAP
    keep1 = pos1 < CAP
    base0 = idx0 * CAPP
    base1 = idx1 * CAPP
    d0 = base0 + jnp.minimum(pos0, CAP - 1.0)
    d1 = base1 + jnp.minimum(pos1, CAP - 1.0)

    d0_ref[...] = d0.astype(jnp.int32)
    d1_ref[...] = d1.astype(jnp.int32)
    s0_ref[...] = (base0 + jnp.minimum(pos0, float(CAP))).astype(jnp.int32)
    s1_ref[...] = (base1 + jnp.minimum(pos1, float(CAP))).astype(jnp.int32)
    g0_ref[...] = g0 * keep0.astype(jnp.float32)
    g1_ref[...] = g1 * keep1.astype(jnp.float32)


def _router(x, Wg):
    blk = pl.BlockSpec((B, 1), lambda i: (i, 0))
    iod = jax.ShapeDtypeStruct((N, 1), jnp.int32)
    fod = jax.ShapeDtypeStruct((N, 1), jnp.float32)
    return pl.pallas_call(
        _router_body,
        grid=(NB,),
        in_specs=[
            pl.BlockSpec((B, D), lambda i: (i, 0)),
            pl.BlockSpec((D, E), lambda i: (0, 0)),
        ],
        out_specs=[blk] * 6,
        out_shape=[iod, iod, iod, iod, fod, fod],
        scratch_shapes=[pltpu.VMEM((1, E), jnp.float32),
                        pltpu.VMEM((B, B), jnp.float32)],
    )(x, Wg)


DCH = 128
DNCH = (N // NW) // DCH


def _dispatch_body(x_hbm, s0_hbm, s1_hbm, disp_hbm,
                   idx_v, xb_v, sem_in, sem_out):
    wid = lax.axis_index("s") * NC + lax.axis_index("c")
    base = wid * (N // NW)

    cps = [pltpu.async_copy(s0_hbm.at[pl.ds(base, DCH)], idx_v.at[0],
                            sem_in),
           pltpu.async_copy(s1_hbm.at[pl.ds(base, DCH)], idx_v.at[1],
                            sem_in),
           pltpu.async_copy(x_hbm.at[pl.ds(base, DCH)], xb_v, sem_in)]
    for cp in cps:
        cp.wait()
    o0 = pltpu.async_copy(xb_v, disp_hbm.at[idx_v.at[0]], sem_out)
    o1 = pltpu.async_copy(xb_v, disp_hbm.at[idx_v.at[1]], sem_out)
    o0.wait()
    o1.wait()


def _dispatch(x, s0, s1):
    mesh = plsc.VectorSubcoreMesh(core_axis_name="c", subcore_axis_name="s")
    f = functools.partial(
        pl.kernel,
        mesh=mesh,
        compiler_params=pltpu.CompilerParams(needs_layout_passes=False),
        out_type=jax.ShapeDtypeStruct((NSLOT, D), jnp.float32),
        scratch_types=[
            pltpu.VMEM((2, DCH), jnp.int32),
            pltpu.VMEM((DCH, D), jnp.float32),
            pltpu.SemaphoreType.DMA,
            pltpu.SemaphoreType.DMA,
        ],
    )(_dispatch_body)
    return f(x, s0, s1)


def _expert_body(disp_ref, we_ref, be_ref, y_ref):
    a = disp_ref[0]
    w = we_ref[0]
    y_ref[0] = (jnp.dot(a, w, preferred_element_type=jnp.float32)
                + be_ref[0])


def _expert_mm(disp, We, be3):
    return pl.pallas_call(
        _expert_body,
        grid=(E,),
        in_specs=[
            pl.BlockSpec((1, CAPP, D), lambda e: (e, 0, 0)),
            pl.BlockSpec((1, D, D), lambda e: (e, 0, 0)),
            pl.BlockSpec((1, 1, D), lambda e: (e, 0, 0)),
        ],
        out_specs=pl.BlockSpec((1, CAPP, D), lambda e: (e, 0, 0)),
        out_shape=jax.ShapeDtypeStruct((E, CAPP, D), jnp.float32),
    )(disp, We, be3)


def _combine_body(y_hbm, d0_hbm, d1_hbm, g0_hbm, g1_hbm, out_hbm,
                  d0_v, d1_v, g0_v, g1_v,
                  b0a_v, b1a_v, b0b_v, b1b_v, b0c_v, b1c_v, oba_v, obb_v,
                  sga, sgb, sgc, sw):
    wid = lax.axis_index("s") * NC + lax.axis_index("c")
    base = wid * TPT
    nch = TPT // CCH

    cps = [pltpu.async_copy(d0_hbm.at[pl.ds(base, TPT)], d0_v, sw),
           pltpu.async_copy(d1_hbm.at[pl.ds(base, TPT)], d1_v, sw),
           pltpu.async_copy(g0_hbm.at[pl.ds(base, TPT)], g0_v, sw),
           pltpu.async_copy(g1_hbm.at[pl.ds(base, TPT)], g1_v, sw)]
    for cp in cps:
        cp.wait()

    b0s = [b0a_v, b0b_v, b0c_v]
    b1s = [b1a_v, b1b_v, b1c_v]
    obs = [oba_v, obb_v]
    sgs = [sga, sgb, sgc]

    def gathers(ch, k):
        t0 = ch * CCH
        pltpu.async_copy(y_hbm.at[d0_v.at[pl.ds(t0, CCH)]], b0s[k], sgs[k])
        pltpu.async_copy(y_hbm.at[d1_v.at[pl.ds(t0, CCH)]], b1s[k], sgs[k])

    gathers(0, 0)
    gathers(1, 1)
    for ch in range(nch):
        k = ch % 3
        ko = ch % 2
        if ch + 2 < nch:
            gathers(ch + 2, (ch + 2) % 3)
        pltpu.make_async_copy(y_hbm.at[d0_v.at[pl.ds(0, CCH)]],
                              b0s[k], sgs[k]).wait()
        pltpu.make_async_copy(y_hbm.at[d1_v.at[pl.ds(0, CCH)]],
                              b1s[k], sgs[k]).wait()
        if ch >= 2:
            pltpu.make_async_copy(obs[ko], out_hbm.at[pl.ds(0, CCH)],
                                  sw).wait()
        t0 = ch * CCH

        def tok_body(tt, _):
            for u in range(2):
                t = tt * 2 + u
                bcast = jnp.zeros((L,), jnp.int32) + (t0 + t)
                gt0 = plsc.load_gather(g0_v, [bcast])
                gt1 = plsc.load_gather(g1_v, [bcast])
                for j in range(D // L):
                    sl = pl.ds(j * L, L)
                    obs[ko][t, sl] = (b0s[k][t, sl] * gt0
                                      + b1s[k][t, sl] * gt1)
            return 0
        lax.fori_loop(0, CCH // 2, tok_body, 0)
        pltpu.async_copy(obs[ko], out_hbm.at[pl.ds(base + t0, CCH)], sw)
    pltpu.make_async_copy(obs[0], out_hbm.at[pl.ds(0, CCH)], sw).wait()
    pltpu.make_async_copy(obs[0], out_hbm.at[pl.ds(0, CCH)], sw).wait()


def _combine(y, d0, d1, g0, g1):
    mesh = plsc.VectorSubcoreMesh(core_axis_name="c", subcore_axis_name="s")
    f = functools.partial(
        pl.kernel,
        mesh=mesh,
        compiler_params=pltpu.CompilerParams(needs_layout_passes=False),
        out_type=jax.ShapeDtypeStruct((N, D), jnp.float32),
        scratch_types=[
            pltpu.VMEM((TPT,), jnp.int32),
            pltpu.VMEM((TPT,), jnp.int32),
            pltpu.VMEM((TPT,), jnp.float32),
            pltpu.VMEM((TPT,), jnp.float32),
            pltpu.VMEM((CCH, D), jnp.float32),
            pltpu.VMEM((CCH, D), jnp.float32),
            pltpu.VMEM((CCH, D), jnp.float32),
            pltpu.VMEM((CCH, D), jnp.float32),
            pltpu.VMEM((CCH, D), jnp.float32),
            pltpu.VMEM((CCH, D), jnp.float32),
            pltpu.VMEM((CCH, D), jnp.float32),
            pltpu.VMEM((CCH, D), jnp.float32),
            pltpu.SemaphoreType.DMA,
            pltpu.SemaphoreType.DMA,
            pltpu.SemaphoreType.DMA,
            pltpu.SemaphoreType.DMA,
        ],
    )(_combine_body)
    return f(y, d0, d1, g0, g1)


def kernel(hidden_states, Wg, We, be):
    x = hidden_states
    d0, d1, s0, s1, g0, g1 = _router(x, Wg)
    d0 = d0.reshape(N)
    d1 = d1.reshape(N)
    s0 = s0.reshape(N)
    s1 = s1.reshape(N)
    g0 = g0.reshape(N)
    g1 = g1.reshape(N)

    disp = _dispatch(x, s0, s1)
    y = _expert_mm(disp.reshape(E, CAPP, D), We, be.reshape(E, 1, D))
    return _combine(y.reshape(NSLOT, D), d0, d1, g0, g1)


# speedup vs baseline: 1.3256x; 1.0016x over previous
import functools

import jax
import jax.numpy as jnp
from jax import lax
from jax.experimental import pallas as pl
from jax.experimental.pallas import tpu as pltpu
from jax.experimental.pallas import tpu_sc as plsc

E = 64
K = 2
D = 768
N = 4096
CAP = 160
CAPP = CAP + 8
NSLOT = E * CAPP

B = 512
NB = N // B

NC = 2
NS = 16
NW = NC * NS
L = 16

TPT = N // NW
CCH = 16


def _router_body(x_ref, wg_ref, d0_ref, d1_ref, s0_ref, s1_ref,
                 g0_ref, g1_ref, carry_ref, tril_ref):
    i = pl.program_id(0)

    @pl.when(i == 0)
    def _():
        carry_ref[...] = jnp.zeros_like(carry_ref)
        rr = lax.broadcasted_iota(jnp.int32, (B, B), 0)
        cc = lax.broadcasted_iota(jnp.int32, (B, B), 1)
        tril_ref[...] = jnp.where(cc < rr, 1.0, 0.0)

    x = x_ref[...]
    wg = wg_ref[...]
    logits = jnp.dot(x, wg, preferred_element_type=jnp.float32)

    re = lax.broadcasted_iota(jnp.int32, (E, E), 0)
    ce = lax.broadcasted_iota(jnp.int32, (E, E), 1)
    ut = jnp.where(re < ce, 1.0, 0.0)
    iota_col = lax.broadcasted_iota(jnp.int32, (E, 1), 0).astype(jnp.float32)
    ones_col = jnp.full((E, 1), 1.0, jnp.float32)

    def first_max(v):
        m = jnp.max(v, axis=1, keepdims=True)
        eq = jnp.where(v == m, 1.0, 0.0)
        before = jnp.dot(eq, ut, preferred_element_type=jnp.float32)
        oh = eq * jnp.where(before == 0.0, 1.0, 0.0)
        idx = jnp.dot(oh, iota_col, preferred_element_type=jnp.float32)
        return m, oh, idx

    m0, oh0, idx0 = first_max(logits)
    masked = jnp.where(oh0 > 0.0, -jnp.inf, logits)
    m1, oh1, idx1 = first_max(masked)

    t = jnp.exp(m1 - m0)
    g0 = 1.0 / (1.0 + t)
    g1 = 1.0 - g0

    ohsum = oh0 + oh1
    cnt_before = (jnp.dot(tril_ref[...], ohsum,
                          preferred_element_type=jnp.float32)
                  + carry_ref[...])
    carry_ref[...] = carry_ref[...] + jnp.sum(ohsum, axis=0, keepdims=True)

    pos0 = jnp.dot(oh0 * cnt_before, ones_col,
                   preferred_element_type=jnp.float32)
    pos1 = jnp.dot(oh1 * cnt_before, ones_col,
                   preferred_element_type=jnp.float32)
    keep0 = pos0 < CAP
    keep1 = pos1 < CAP
    base0 = idx0 * CAPP
    base1 = idx1 * CAPP
    d0 = base0 + jnp.minimum(pos0, CAP - 1.0)
    d1 = base1 + jnp.minimum(pos1, CAP - 1.0)

    d0_ref[...] = d0.astype(jnp.int32)
    d1_ref[...] = d1.astype(jnp.int32)
    s0_ref[...] = (base0 + jnp.minimum(pos0, float(CAP))).astype(jnp.int32)
    s1_ref[...] = (base1 + jnp.minimum(pos1, float(CAP))).astype(jnp.int32)
    g0_ref[...] = g0 * keep0.astype(jnp.float32)
    g1_ref[...] = g1 * keep1.astype(jnp.float32)


def _router(x, Wg):
    blk = pl.BlockSpec((B, 1), lambda i: (i, 0))
    iod = jax.ShapeDtypeStruct((N, 1), jnp.int32)
    fod = jax.ShapeDtypeStruct((N, 1), jnp.float32)
    return pl.pallas_call(
        _router_body,
        grid=(NB,),
        in_specs=[
            pl.BlockSpec((B, D), lambda i: (i, 0)),
            pl.BlockSpec((D, E), lambda i: (0, 0)),
        ],
        out_specs=[blk] * 6,
        out_shape=[iod, iod, iod, iod, fod, fod],
        scratch_shapes=[pltpu.VMEM((1, E), jnp.float32),
                        pltpu.VMEM((B, B), jnp.float32)],
    )(x, Wg)


DCH = 128
DNCH = (N // NW) // DCH


def _dispatch_body(x_hbm, s0_hbm, s1_hbm, disp_hbm,
                   idx_v, xb_v, sem_in, sem_out):
    wid = lax.axis_index("s") * NC + lax.axis_index("c")
    base = wid * (N // NW)

    cps = [pltpu.async_copy(s0_hbm.at[pl.ds(base, DCH)], idx_v.at[0],
                            sem_in),
           pltpu.async_copy(s1_hbm.at[pl.ds(base, DCH)], idx_v.at[1],
                            sem_in),
           pltpu.async_copy(x_hbm.at[pl.ds(base, DCH)], xb_v, sem_in)]
    for cp in cps:
        cp.wait()
    o0 = pltpu.async_copy(xb_v, disp_hbm.at[idx_v.at[0]], sem_out)
    o1 = pltpu.async_copy(xb_v, disp_hbm.at[idx_v.at[1]], sem_out)
    o0.wait()
    o1.wait()


def _dispatch(x, s0, s1):
    mesh = plsc.VectorSubcoreMesh(core_axis_name="c", subcore_axis_name="s")
    f = functools.partial(
        pl.kernel,
        mesh=mesh,
        compiler_params=pltpu.CompilerParams(needs_layout_passes=False),
        out_type=jax.ShapeDtypeStruct((NSLOT, D), jnp.float32),
        scratch_types=[
            pltpu.VMEM((2, DCH), jnp.int32),
            pltpu.VMEM((DCH, D), jnp.float32),
            pltpu.SemaphoreType.DMA,
            pltpu.SemaphoreType.DMA,
        ],
    )(_dispatch_body)
    return f(x, s0, s1)


def _expert_body(disp_ref, we_ref, be_ref, y_ref):
    a = disp_ref[0]
    w = we_ref[0]
    y_ref[0] = (jnp.dot(a, w, preferred_element_type=jnp.float32)
                + be_ref[0])


def _expert_mm(disp, We, be3):
    return pl.pallas_call(
        _expert_body,
        grid=(E,),
        in_specs=[
            pl.BlockSpec((1, CAPP, D), lambda e: (e, 0, 0)),
            pl.BlockSpec((1, D, D), lambda e: (e, 0, 0)),
            pl.BlockSpec((1, 1, D), lambda e: (e, 0, 0)),
        ],
        out_specs=pl.BlockSpec((1, CAPP, D), lambda e: (e, 0, 0)),
        out_shape=jax.ShapeDtypeStruct((E, CAPP, D), jnp.float32),
    )(disp, We, be3)


def _combine_body(y_hbm, d0_hbm, d1_hbm, g0_hbm, g1_hbm, out_hbm,
                  d0_v, d1_v, g0_v, g1_v,
                  b0a_v, b1a_v, b0b_v, b1b_v, b0c_v, b1c_v, oba_v, obb_v,
                  sga, sgb, sgc, sw):
    wid = lax.axis_index("s") * NC + lax.axis_index("c")
    base = wid * TPT
    nch = TPT // CCH

    cps = [pltpu.async_copy(d0_hbm.at[pl.ds(base, TPT)], d0_v, sw),
           pltpu.async_copy(d1_hbm.at[pl.ds(base, TPT)], d1_v, sw),
           pltpu.async_copy(g0_hbm.at[pl.ds(base, TPT)], g0_v, sw),
           pltpu.async_copy(g1_hbm.at[pl.ds(base, TPT)], g1_v, sw)]
    for cp in cps:
        cp.wait()

    b0s = [b0a_v, b0b_v, b0c_v]
    b1s = [b1a_v, b1b_v, b1c_v]
    obs = [oba_v, obb_v]
    sgs = [sga, sgb, sgc]

    def gathers(ch, k):
        t0 = ch * CCH
        pltpu.async_copy(y_hbm.at[d0_v.at[pl.ds(t0, CCH)]], b0s[k], sgs[k])
        pltpu.async_copy(y_hbm.at[d1_v.at[pl.ds(t0, CCH)]], b1s[k], sgs[k])

    gathers(0, 0)
    gathers(1, 1)
    for ch in range(nch):
        k = ch % 3
        ko = ch % 2
        if ch + 2 < nch:
            gathers(ch + 2, (ch + 2) % 3)
        pltpu.make_async_copy(y_hbm.at[d0_v.at[pl.ds(0, CCH)]],
                              b0s[k], sgs[k]).wait()
        pltpu.make_async_copy(y_hbm.at[d1_v.at[pl.ds(0, CCH)]],
                              b1s[k], sgs[k]).wait()
        if ch >= 2:
            pltpu.make_async_copy(obs[ko], out_hbm.at[pl.ds(0, CCH)],
                                  sw).wait()
        t0 = ch * CCH

        def tok_body(tt, _):
            for u in range(2):
                t = tt * 2 + u
                bcast = jnp.zeros((L,), jnp.int32) + (t0 + t)
                gt0 = plsc.load_gather(g0_v, [bcast])
                gt1 = plsc.load_gather(g1_v, [bcast])
                for j in range(D // L):
                    sl = pl.ds(j * L, L)
                    obs[ko][t, sl] = (b0s[k][t, sl] * gt0
                                      + b1s[k][t, sl] * gt1)
            return 0
        lax.fori_loop(0, CCH // 2, tok_body, 0)
        pltpu.async_copy(obs[ko], out_hbm.at[pl.ds(base + t0, CCH)], sw)
    pltpu.make_async_copy(obs[0], out_hbm.at[pl.ds(0, CCH)], sw).wait()
    pltpu.make_async_copy(obs[0], out_hbm.at[pl.ds(0, CCH)], sw).wait()


def _combine(y, d0, d1, g0, g1):
    mesh = plsc.VectorSubcoreMesh(core_axis_name="c", subcore_axis_name="s")
    f = functools.partial(
        pl.kernel,
        mesh=mesh,
        compiler_params=pltpu.CompilerParams(needs_layout_passes=False),
        out_type=jax.ShapeDtypeStruct((N, D), jnp.float32),
        scratch_types=[
            pltpu.VMEM((TPT,), jnp.int32),
            pltpu.VMEM((TPT,), jnp.int32),
            pltpu.VMEM((TPT,), jnp.float32),
            pltpu.VMEM((TPT,), jnp.float32),
            pltpu.VMEM((CCH, D), jnp.float32),
            pltpu.VMEM((CCH, D), jnp.float32),
            pltpu.VMEM((CCH, D), jnp.float32),
            pltpu.VMEM((CCH, D), jnp.float32),
            pltpu.VMEM((CCH, D), jnp.float32),
            pltpu.VMEM((CCH, D), jnp.float32),
            pltpu.VMEM((CCH, D), jnp.float32),
            pltpu.VMEM((CCH, D), jnp.float32),
            pltpu.SemaphoreType.DMA,
            pltpu.SemaphoreType.DMA,
            pltpu.SemaphoreType.DMA,
            pltpu.SemaphoreType.DMA,
        ],
    )(_combine_body)
    return f(y, d0, d1, g0, g1)


def kernel(hidden_states, Wg, We, be):
    x = hidden_states
    d0, d1, s0, s1, g0, g1 = _router(x, Wg)
    d0 = d0.reshape(N)
    d1 = d1.reshape(N)
    s0 = s0.reshape(N)
    s1 = s1.reshape(N)
    g0 = g0.reshape(N)
    g1 = g1.reshape(N)

    disp = _dispatch(x, s0, s1)
    y = _expert_mm(disp.reshape(E, CAPP, D), We, be.reshape(E, 1, D))
    return _combine(y.reshape(NSLOT, D), d0, d1, g0, g1)
